# baseline (speedup 1.0000x reference)
import functools

import jax
import jax.numpy as jnp
from jax import lax
from jax.experimental import pallas as pl
from jax.experimental.pallas import tpu as pltpu
from jax.experimental.pallas import tpu_sc as plsc

N = 100000
G = 64
D = 3
MIN_SCALE_CONST = 0.1

_INFO = plsc.get_sparse_core_info()
_NC = _INFO.num_cores
_NS = _INFO.num_subcores
_NW = _NC * _NS

_CH = 3136
_CH_LAST = N - (_NW - 1) * _CH
_IT = _CH // 16
_IT_LAST = _CH_LAST // 16

_XSUB = 8


def _newton_sqrt(s):
    i = plsc.bitcast(s, jnp.int32)
    y = plsc.bitcast(lax.shift_right_logical(i, 1) + 0x1FBD1DF5, jnp.float32)
    for _ in range(2):
        y = 0.5 * (y + s / y)
    return y


def _sc_body(pred_hbm, targ_hbm, batch_hbm, x8_hbm,
             sums_hbm, cnts_hbm, force_hbm,
             pred_v, targ_v, batch_v, idx_v, xr_v, acc_v, cnt_v, force_v,
             sem_in, sem_g):
    wid = lax.axis_index("s") * _NC + lax.axis_index("c")
    r0 = wid * _CH

    z = jnp.zeros((16,), jnp.float32)
    for k in range(G // 16):
        acc_v[pl.ds(16 * k, 16)] = z
        cnt_v[pl.ds(16 * k, 16)] = z

    lane = lax.iota(jnp.int32, 16)
    ones = jnp.ones((16,), jnp.float32)

    def _run(chunk, iters, unroll):
        cps = []
        for c in range(D):
            cps.append(pltpu.async_copy(
                pred_hbm.at[pl.ds(c * N + r0, chunk)],
                pred_v.at[pl.ds(c * _CH, chunk)], sem_in))
            cps.append(pltpu.async_copy(
                targ_hbm.at[pl.ds(c * N + r0, chunk)],
                targ_v.at[pl.ds(c * _CH, chunk)], sem_in))
        cps.append(pltpu.async_copy(batch_hbm.at[pl.ds(r0, chunk)],
                                    batch_v.at[pl.ds(0, chunk)], sem_in))

        def build_idx(i, carry):
            base = i * 16
            idx_v[pl.ds(base, 16)] = \
                jnp.minimum(r0 + base + lane, N - 1) * 16
            return carry

        lax.fori_loop(0, _IT, build_idx, 0, unroll=4)
        g = pltpu.async_copy(x8_hbm.at[idx_v], xr_v, sem_g)
        for cp in cps:
            cp.wait()
        g.wait()

        c3 = jnp.full((16,), 3, jnp.int32)
        c4 = jnp.full((16,), 4, jnp.int32)

        def body(i, facc):
            base = i * 16
            rows = base + lane
            b16 = batch_v[pl.ds(base, 16)]
            plsc.addupdate_scatter(cnt_v, [b16], ones)
            e = jnp.zeros((16,), jnp.float32)
            for c in range(D):
                p = pred_v[pl.ds(c * _CH + base, 16)]
                t = targ_v[pl.ds(c * _CH + base, 16)]
                e = e + jnp.abs(p - t)
            plsc.addupdate_scatter(acc_v, [b16], e)
            a = plsc.load_gather(xr_v, [rows, c3])
            b = plsc.load_gather(xr_v, [rows, c4])
            return facc + _newton_sqrt(a * a + b * b)

        facc = lax.fori_loop(0, iters, body, z, unroll=unroll)
        force_v[pl.ds(0, 16)] = facc

    @pl.when(wid < _NW - 1)
    def _():
        _run(_CH, _IT, 4)

    @pl.when(wid == _NW - 1)
    def _():
        _run(_CH_LAST, _IT_LAST, 2)

    pltpu.sync_copy(acc_v, sums_hbm.at[pl.ds(wid * G, G)])
    pltpu.sync_copy(cnt_v, cnts_hbm.at[pl.ds(wid * G, G)])
    pltpu.sync_copy(force_v, force_hbm.at[pl.ds(wid * 16, 16)])


@jax.jit
def _sc_all(predf, targf, batch, x8):
    mesh = plsc.VectorSubcoreMesh(core_axis_name="c", subcore_axis_name="s")
    f = functools.partial(
        pl.kernel,
        mesh=mesh,
        out_type=(
            jax.ShapeDtypeStruct((_NW * G,), jnp.float32),
            jax.ShapeDtypeStruct((_NW * G,), jnp.float32),
            jax.ShapeDtypeStruct((_NW * 16,), jnp.float32),
        ),
        scratch_types=[
            pltpu.VMEM((_CH * D,), jnp.float32),
            pltpu.VMEM((_CH * D,), jnp.float32),
            pltpu.VMEM((_CH,), jnp.int32),
            pltpu.VMEM((_CH,), jnp.int32),
            pltpu.VMEM((_CH, _XSUB), jnp.float32),
            pltpu.VMEM((G,), jnp.float32),
            pltpu.VMEM((G,), jnp.float32),
            pltpu.VMEM((16,), jnp.float32),
            pltpu.SemaphoreType.DMA,
            pltpu.SemaphoreType.DMA,
        ],
        compiler_params=pltpu.CompilerParams(needs_layout_passes=False,
                                             use_tc_tiling_on_sc=False),
    )(_sc_body)
    return f(predf, targf, batch, x8)


def _fin_body(sums_ref, cnts_ref, force_ref, out_ref):
    seg = jnp.zeros((G,), jnp.float32)
    cnt = jnp.zeros((G,), jnp.float32)
    for w in range(_NW):
        seg = seg + sums_ref[pl.ds(w * G, G)]
        cnt = cnt + cnts_ref[pl.ds(w * G, G)]
    mae = seg / (cnt * float(D))
    scale = jnp.maximum(jnp.sum(force_ref[...]), MIN_SCALE_CONST)
    out_ref[...] = jnp.full((1, 1), jnp.mean(mae) * scale * 100.0,
                            dtype=jnp.float32)


@jax.jit
def _finalize(sums, cnts, force):
    return pl.pallas_call(
        _fin_body,
        out_shape=jax.ShapeDtypeStruct((1, 1), jnp.float32),
    )(sums, cnts, force)


def kernel(pred, target, batch, x):
    batch = batch.astype(jnp.int32)
    predf = jnp.transpose(pred).reshape(-1)
    targf = jnp.transpose(target).reshape(-1)
    x8 = x.reshape(N * 16, _XSUB)
    sums, cnts, force = _sc_all(predf, targf, batch, x8)
    out = _finalize(sums, cnts, force)
    return out[0, 0]


# speedup vs baseline: 1.0757x; 1.0757x over previous
import functools

import jax
import jax.numpy as jnp
from jax import lax
from jax.experimental import pallas as pl
from jax.experimental.pallas import tpu as pltpu
from jax.experimental.pallas import tpu_sc as plsc

N = 100000
G = 64
D = 3
MIN_SCALE_CONST = 0.1

_INFO = plsc.get_sparse_core_info()
_NC = _INFO.num_cores
_NS = _INFO.num_subcores
_NW = _NC * _NS

_CH = 3136
_CH_LAST = N - (_NW - 1) * _CH
_IT = _CH // 16
_IT_LAST = _CH_LAST // 16

_XSUB = 8


def _newton_sqrt(s):
    i = plsc.bitcast(s, jnp.int32)
    y = plsc.bitcast(lax.shift_right_logical(i, 1) + 0x1FBD1DF5, jnp.float32)
    for _ in range(2):
        y = 0.5 * (y + s / y)
    return y


def _sc_body(pred_hbm, targ_hbm, batch_hbm, x8_hbm,
             sums_hbm, cnts_hbm, force_hbm,
             pred_v, targ_v, batch_v, idx_v, xr_v, acc_v, cnt_v, force_v,
             sem_in, sem_g):
    wid = lax.axis_index("s") * _NC + lax.axis_index("c")
    r0 = wid * _CH

    z = jnp.zeros((16,), jnp.float32)
    for k in range(G // 16):
        acc_v[pl.ds(16 * k, 16)] = z
        cnt_v[pl.ds(16 * k, 16)] = z

    lane = lax.iota(jnp.int32, 16)
    ones = jnp.ones((16,), jnp.float32)

    def _run(chunk, iters, unroll):
        cps = []
        for c in range(D):
            cps.append(pltpu.async_copy(
                pred_hbm.at[pl.ds(c * N + r0, chunk)],
                pred_v.at[pl.ds(c * _CH, chunk)], sem_in))
            cps.append(pltpu.async_copy(
                targ_hbm.at[pl.ds(c * N + r0, chunk)],
                targ_v.at[pl.ds(c * _CH, chunk)], sem_in))
        cps.append(pltpu.async_copy(batch_hbm.at[pl.ds(r0, chunk)],
                                    batch_v.at[pl.ds(0, chunk)], sem_in))

        def build_idx(i, carry):
            base = i * 16
            idx_v[pl.ds(base, 16)] = \
                jnp.minimum(r0 + base + lane, N - 1) * 16
            return carry

        lax.fori_loop(0, _IT, build_idx, 0, unroll=4)
        g = pltpu.async_copy(x8_hbm.at[idx_v], xr_v, sem_g)
        for cp in cps:
            cp.wait()

        def body(i, carry):
            base = i * 16
            b16 = batch_v[pl.ds(base, 16)]
            plsc.addupdate_scatter(cnt_v, [b16], ones)
            e = jnp.zeros((16,), jnp.float32)
            for c in range(D):
                p = pred_v[pl.ds(c * _CH + base, 16)]
                t = targ_v[pl.ds(c * _CH + base, 16)]
                e = e + jnp.abs(p - t)
            plsc.addupdate_scatter(acc_v, [b16], e)
            return carry

        lax.fori_loop(0, iters, body, 0, unroll=unroll)
        g.wait()

        c3 = jnp.full((16,), 3, jnp.int32)
        c4 = jnp.full((16,), 4, jnp.int32)

        def fbody(i, facc):
            base = i * 16
            rows = base + lane
            a = plsc.load_gather(xr_v, [rows, c3])
            b = plsc.load_gather(xr_v, [rows, c4])
            return facc + _newton_sqrt(a * a + b * b)

        facc = lax.fori_loop(0, iters, fbody, z, unroll=unroll)
        force_v[pl.ds(0, 16)] = facc

    @pl.when(wid < _NW - 1)
    def _():
        _run(_CH, _IT, 4)

    @pl.when(wid == _NW - 1)
    def _():
        _run(_CH_LAST, _IT_LAST, 2)

    pltpu.sync_copy(acc_v, sums_hbm.at[pl.ds(wid * G, G)])
    pltpu.sync_copy(cnt_v, cnts_hbm.at[pl.ds(wid * G, G)])
    pltpu.sync_copy(force_v, force_hbm.at[pl.ds(wid * 16, 16)])


@jax.jit
def _sc_all(predf, targf, batch, x8):
    mesh = plsc.VectorSubcoreMesh(core_axis_name="c", subcore_axis_name="s")
    f = functools.partial(
        pl.kernel,
        mesh=mesh,
        out_type=(
            jax.ShapeDtypeStruct((_NW * G,), jnp.float32),
            jax.ShapeDtypeStruct((_NW * G,), jnp.float32),
            jax.ShapeDtypeStruct((_NW * 16,), jnp.float32),
        ),
        scratch_types=[
            pltpu.VMEM((_CH * D,), jnp.float32),
            pltpu.VMEM((_CH * D,), jnp.float32),
            pltpu.VMEM((_CH,), jnp.int32),
            pltpu.VMEM((_CH,), jnp.int32),
            pltpu.VMEM((_CH, _XSUB), jnp.float32),
            pltpu.VMEM((G,), jnp.float32),
            pltpu.VMEM((G,), jnp.float32),
            pltpu.VMEM((16,), jnp.float32),
            pltpu.SemaphoreType.DMA,
            pltpu.SemaphoreType.DMA,
        ],
        compiler_params=pltpu.CompilerParams(needs_layout_passes=False,
                                             use_tc_tiling_on_sc=False),
    )(_sc_body)
    return f(predf, targf, batch, x8)


def _fin_body(sums_ref, cnts_ref, force_ref, out_ref):
    seg = jnp.zeros((G,), jnp.float32)
    cnt = jnp.zeros((G,), jnp.float32)
    for w in range(_NW):
        seg = seg + sums_ref[pl.ds(w * G, G)]
        cnt = cnt + cnts_ref[pl.ds(w * G, G)]
    mae = seg / (cnt * float(D))
    scale = jnp.maximum(jnp.sum(force_ref[...]), MIN_SCALE_CONST)
    out_ref[...] = jnp.full((1, 1), jnp.mean(mae) * scale * 100.0,
                            dtype=jnp.float32)


@jax.jit
def _finalize(sums, cnts, force):
    return pl.pallas_call(
        _fin_body,
        out_shape=jax.ShapeDtypeStruct((1, 1), jnp.float32),
    )(sums, cnts, force)


def kernel(pred, target, batch, x):
    batch = batch.astype(jnp.int32)
    predf = jnp.transpose(pred).reshape(-1)
    targf = jnp.transpose(target).reshape(-1)
    x8 = x.reshape(N * 16, _XSUB)
    sums, cnts, force = _sc_all(predf, targf, batch, x8)
    out = _finalize(sums, cnts, force)
    return out[0, 0]
